# baseline (speedup 1.0000x reference)
import functools

import jax
import jax.numpy as jnp
from jax import lax
from jax.experimental import pallas as pl
from jax.experimental.pallas import tpu as pltpu
from jax.experimental.pallas import tpu_sc as plsc

_B, _S, _H, _E = 2, 2048, 768, 8
_N = _B * _S
_TILE = 512
_NT = _N // _TILE
_T1 = 512
_NT1 = _N // _T1
_NC, _NS, _L = 2, 16, 16
_NW = _NC * _NS
_TPW = _N // _NW
_WPB = _S // _TPW


def _router_body(x_ref, w1_ref, b1_ref, w2_ref, b2_ref,
                 pos_ref, offs_ref, run_ref, triu_ref, eid_s, rank_s):
    p = pl.program_id(0)
    t = pl.program_id(1)

    @pl.when(jnp.logical_and(p == 0, t == 0))
    def _init():
        run_ref[...] = jnp.zeros_like(run_ref)
        ri = lax.broadcasted_iota(jnp.int32, (_T1, _T1), 0)
        ci = lax.broadcasted_iota(jnp.int32, (_T1, _T1), 1)
        triu_ref[...] = (ri < ci).astype(jnp.float32)

    rowi = lax.broadcasted_iota(jnp.int32, (_E, _T1), 0)
    er = lax.broadcasted_iota(jnp.int32, (_E, _E), 0)
    ec = lax.broadcasted_iota(jnp.int32, (_E, _E), 1)
    ltm = (ec < er).astype(jnp.float32)

    @pl.when(p == 0)
    def _pass0():
        x = x_ref[0]
        h = jnp.dot(x, w1_ref[...], preferred_element_type=jnp.float32)
        h = jnp.maximum(h + b1_ref[...], 0.0)
        s = jnp.dot(h, w2_ref[...], preferred_element_type=jnp.float32)
        s = s + b2_ref[...]

        st = s.T
        smax = jnp.max(st, axis=0, keepdims=True)
        eid = jnp.min(jnp.where(st == smax, rowi, _E), axis=0)
        eid_s[t, 0, :] = eid

        onehot = (rowi == eid[None, :]).astype(jnp.float32)
        excl = jnp.dot(onehot, triu_ref[...],
                       preferred_element_type=jnp.float32)
        run = run_ref[...]
        rank = jnp.sum(onehot * (excl + run), axis=0)
        rank_s[t, 0, :] = rank.astype(jnp.int32)

        counts = jnp.dot(onehot, jnp.ones((_T1, 1), jnp.float32),
                         preferred_element_type=jnp.float32)
        run_ref[...] = run + counts

    offs = jnp.dot(ltm, run_ref[...], preferred_element_type=jnp.float32,
                   precision=lax.Precision.HIGHEST)
    offs_ref[...] = offs

    @pl.when(p == 1)
    def _pass1():
        eid = eid_s[t, 0, :]
        rank = rank_s[t, 0, :]
        onehot = (rowi == eid[None, :]).astype(jnp.float32)
        posf = jnp.sum(onehot * offs, axis=0)
        pos_ref[0, 0, :] = rank + posf.astype(jnp.int32)


def _router_call(hs, w1, b1, w2, b2):
    nb = _S // _T1

    def _xmap(p, t):
        tt = jnp.where(p == 0, t, _NT1 - 1)
        return (tt // nb, tt % nb, 0)

    return pl.pallas_call(
        _router_body,
        grid=(2, _NT1),
        in_specs=[
            pl.BlockSpec((1, _T1, _H), _xmap),
            pl.BlockSpec((_H, _H // 2), lambda p, t: (0, 0)),
            pl.BlockSpec((1, _H // 2), lambda p, t: (0, 0)),
            pl.BlockSpec((_H // 2, _E), lambda p, t: (0, 0)),
            pl.BlockSpec((1, _E), lambda p, t: (0, 0)),
        ],
        out_specs=[
            pl.BlockSpec((1, 1, _T1), lambda p, t: (t, 0, 0)),
            pl.BlockSpec((_E, 1), lambda p, t: (0, 0)),
        ],
        out_shape=[
            jax.ShapeDtypeStruct((_NT1, 1, _T1), jnp.int32),
            jax.ShapeDtypeStruct((_E, 1), jnp.float32),
        ],
        scratch_shapes=[pltpu.VMEM((_E, 1), jnp.float32),
                        pltpu.VMEM((_T1, _T1), jnp.float32),
                        pltpu.VMEM((_NT1, 1, _T1), jnp.int32),
                        pltpu.VMEM((_NT1, 1, _T1), jnp.int32)],
    )(hs, w1, b1, w2, b2)


def _dispatch_body(x_hbm, pos_hbm, perm_hbm, pos_v, rows_v, sem_rows):
    wid = lax.axis_index("s") * _NC + lax.axis_index("c")
    base = wid * _TPW
    b, s0 = wid // _WPB, (wid % _WPB) * _TPW
    rows_cp = pltpu.async_copy(x_hbm.at[b, pl.ds(s0, _TPW)], rows_v, sem_rows)
    pltpu.sync_copy(pos_hbm.at[pl.ds(base, _TPW)], pos_v)
    rows_cp.wait()
    pltpu.sync_copy(rows_v, perm_hbm.at[pos_v])


@functools.cache
def _dispatch_call():
    return pl.kernel(
        _dispatch_body,
        out_type=jax.ShapeDtypeStruct((_N, _H), jnp.float32),
        mesh=plsc.VectorSubcoreMesh(core_axis_name="c", subcore_axis_name="s"),
        scratch_types=[
            pltpu.VMEM((_TPW,), jnp.int32),
            pltpu.VMEM((_TPW, _H), jnp.float32),
            pltpu.SemaphoreType.DMA,
        ],
    )


def _moe_body(offs_ref, x_ref, w_ref, b_ref, out_ref):
    t = pl.program_id(0)
    base = t * _TILE
    rows = base + lax.broadcasted_iota(jnp.int32, (_TILE, 1), 0)
    xbf = x_ref[...].astype(jnp.bfloat16)
    for e in range(_E):
        start = offs_ref[e]
        end = offs_ref[e + 1] if e + 1 < _E else _N

        @pl.when(jnp.logical_and(start < base + _TILE, end > base))
        def _apply(e=e, start=start, end=end):
            m = jnp.logical_and(rows >= start, rows < end)
            part = jnp.dot(xbf, w_ref[e], preferred_element_type=jnp.float32)
            out_ref[...] = jnp.where(m, part + b_ref[e], out_ref[...])


def _moe_call(offs, xs, w, b):
    grid_spec = pltpu.PrefetchScalarGridSpec(
        num_scalar_prefetch=1,
        grid=(_NT,),
        in_specs=[
            pl.BlockSpec((_TILE, _H), lambda t, offs: (t, 0)),
            pl.BlockSpec((_E, _H, _H), lambda t, offs: (0, 0, 0)),
            pl.BlockSpec((_E, _H), lambda t, offs: (0, 0)),
        ],
        out_specs=pl.BlockSpec((_TILE, _H), lambda t, offs: (t, 0)),
    )
    return pl.pallas_call(
        _moe_body,
        grid_spec=grid_spec,
        out_shape=jax.ShapeDtypeStruct((_N, _H), jnp.float32),
    )(offs, xs, w, b)


def _combine_body(y_hbm, pos_hbm, out_hbm, pos_v, rows_v, sem):
    wid = lax.axis_index("s") * _NC + lax.axis_index("c")
    base = wid * _TPW
    b, s0 = wid // _WPB, (wid % _WPB) * _TPW
    pltpu.sync_copy(pos_hbm.at[pl.ds(base, _TPW)], pos_v)
    pltpu.async_copy(y_hbm.at[pos_v], rows_v, sem).wait()
    pltpu.sync_copy(rows_v, out_hbm.at[b, pl.ds(s0, _TPW)])


@functools.cache
def _combine_call():
    return pl.kernel(
        _combine_body,
        out_type=jax.ShapeDtypeStruct((_B, _S, _H), jnp.float32),
        mesh=plsc.VectorSubcoreMesh(core_axis_name="c", subcore_axis_name="s"),
        scratch_types=[
            pltpu.VMEM((_TPW,), jnp.int32),
            pltpu.VMEM((_TPW, _H), jnp.float32),
            pltpu.SemaphoreType.DMA,
        ],
    )


def kernel(hidden_states, W1, b1, W2, b2, expert_W, expert_b):
    pos3, offs_f = _router_call(
        hidden_states, W1, b1.reshape(1, -1), W2, b2.reshape(1, -1))
    offs = offs_f.reshape(_E).astype(jnp.int32)
    pos = pos3.reshape(_N)
    perm_x = _dispatch_call()(hidden_states, pos)
    y = _moe_call(offs, perm_x, expert_W.astype(jnp.bfloat16), expert_b)
    return _combine_call()(y, pos)


# speedup vs baseline: 1.0149x; 1.0149x over previous
import functools

import jax
import jax.numpy as jnp
from jax import lax
from jax.experimental import pallas as pl
from jax.experimental.pallas import tpu as pltpu
from jax.experimental.pallas import tpu_sc as plsc

_B, _S, _H, _E = 2, 2048, 768, 8
_N = _B * _S
_TILE = 512
_NT = _N // _TILE
_T1 = 512
_NT1 = _N // _T1
_NC, _NS, _L = 2, 16, 16
_NW = _NC * _NS
_TPW = _N // _NW
_WPB = _S // _TPW


def _router_body(x_ref, w1_ref, b1_ref, w2_ref, b2_ref,
                 eid_ref, rank_ref, offs_ref, run_ref, triu_ref):
    t = pl.program_id(0)

    @pl.when(t == 0)
    def _init():
        run_ref[...] = jnp.zeros_like(run_ref)
        ri = lax.broadcasted_iota(jnp.int32, (_T1, _T1), 0)
        ci = lax.broadcasted_iota(jnp.int32, (_T1, _T1), 1)
        triu_ref[...] = (ri < ci).astype(jnp.float32)

    x = x_ref[0]
    h = jnp.dot(x, w1_ref[...], preferred_element_type=jnp.float32)
    h = jnp.maximum(h + b1_ref[...], 0.0)
    s = jnp.dot(h, w2_ref[...], preferred_element_type=jnp.float32)
    s = s + b2_ref[...]

    st = s.T
    smax = jnp.max(st, axis=0, keepdims=True)
    rowi = lax.broadcasted_iota(jnp.int32, (_E, _T1), 0)
    eid = jnp.min(jnp.where(st == smax, rowi, _E), axis=0)
    eid_ref[0, 0, :] = eid

    onehot = (rowi == eid[None, :]).astype(jnp.float32)
    excl = jnp.dot(onehot, triu_ref[...], preferred_element_type=jnp.float32)
    run = run_ref[...]
    rank = jnp.sum(onehot * (excl + run), axis=0)
    rank_ref[0, 0, :] = rank.astype(jnp.int32)

    counts = jnp.dot(onehot, jnp.ones((_T1, 1), jnp.float32),
                     preferred_element_type=jnp.float32)
    new_run = run + counts
    run_ref[...] = new_run

    er = lax.broadcasted_iota(jnp.int32, (_E, _E), 0)
    ec = lax.broadcasted_iota(jnp.int32, (_E, _E), 1)
    ltm = (ec < er).astype(jnp.float32)
    offs = jnp.dot(ltm, new_run, preferred_element_type=jnp.float32,
                   precision=lax.Precision.HIGHEST)
    offs_ref[...] = offs


def _router_call(hs, w1, b1, w2, b2):
    return pl.pallas_call(
        _router_body,
        grid=(_NT1,),
        in_specs=[
            pl.BlockSpec((1, _T1, _H), lambda t: (t // (_S // _T1),
                                                  t % (_S // _T1), 0)),
            pl.BlockSpec((_H, _H // 2), lambda t: (0, 0)),
            pl.BlockSpec((1, _H // 2), lambda t: (0, 0)),
            pl.BlockSpec((_H // 2, _E), lambda t: (0, 0)),
            pl.BlockSpec((1, _E), lambda t: (0, 0)),
        ],
        out_specs=[
            pl.BlockSpec((1, 1, _T1), lambda t: (t, 0, 0)),
            pl.BlockSpec((1, 1, _T1), lambda t: (t, 0, 0)),
            pl.BlockSpec((_E, 1), lambda t: (0, 0)),
        ],
        out_shape=[
            jax.ShapeDtypeStruct((_NT1, 1, _T1), jnp.int32),
            jax.ShapeDtypeStruct((_NT1, 1, _T1), jnp.int32),
            jax.ShapeDtypeStruct((_E, 1), jnp.float32),
        ],
        scratch_shapes=[pltpu.VMEM((_E, 1), jnp.float32),
                        pltpu.VMEM((_T1, _T1), jnp.float32)],
    )(hs, w1, b1, w2, b2)


def _dispatch_body(x_hbm, eid_hbm, rank_hbm, offs_hbm,
                   perm_hbm, pos_hbm,
                   eid_v, rank_v, offs_v, pos_v, rows_v, sem_rows, sem_idx):
    wid = lax.axis_index("s") * _NC + lax.axis_index("c")
    base = wid * _TPW
    b, s0 = wid // _WPB, (wid % _WPB) * _TPW
    rows_cp = pltpu.async_copy(x_hbm.at[b, pl.ds(s0, _TPW)], rows_v, sem_rows)
    cp_e = pltpu.async_copy(eid_hbm.at[pl.ds(base, _TPW)], eid_v, sem_idx)
    cp_r = pltpu.async_copy(rank_hbm.at[pl.ds(base, _TPW)], rank_v, sem_idx)
    cp_o = pltpu.async_copy(offs_hbm, offs_v, sem_idx)
    cp_e.wait()
    cp_r.wait()
    cp_o.wait()
    offs_s = [offs_v[pl.ds(e * _L, _L)] for e in range(_E)]
    for i in range(_TPW // _L):
        ev = eid_v[pl.ds(i * _L, _L)]
        pos = rank_v[pl.ds(i * _L, _L)]
        for e in range(_E):
            pos = jnp.where(ev == e, pos + offs_s[e], pos)
        pos_v[pl.ds(i * _L, _L)] = pos
    pltpu.sync_copy(pos_v, pos_hbm.at[pl.ds(base, _TPW)])
    rows_cp.wait()
    pltpu.sync_copy(rows_v, perm_hbm.at[pos_v])


@functools.cache
def _dispatch_call():
    return pl.kernel(
        _dispatch_body,
        out_type=[
            jax.ShapeDtypeStruct((_N, _H), jnp.float32),
            jax.ShapeDtypeStruct((_N,), jnp.int32),
        ],
        mesh=plsc.VectorSubcoreMesh(core_axis_name="c", subcore_axis_name="s"),
        scratch_types=[
            pltpu.VMEM((_TPW,), jnp.int32),
            pltpu.VMEM((_TPW,), jnp.int32),
            pltpu.VMEM((_E * _L,), jnp.int32),
            pltpu.VMEM((_TPW,), jnp.int32),
            pltpu.VMEM((_TPW, _H), jnp.float32),
            pltpu.SemaphoreType.DMA,
            pltpu.SemaphoreType.DMA,
        ],
    )


def _moe_body(offs_ref, x_ref, w_ref, b_ref, out_ref):
    t = pl.program_id(0)
    base = t * _TILE
    rows = base + lax.broadcasted_iota(jnp.int32, (_TILE, 1), 0)
    xbf = x_ref[...].astype(jnp.bfloat16)
    for e in range(_E):
        start = offs_ref[e]
        end = offs_ref[e + 1] if e + 1 < _E else _N

        @pl.when(jnp.logical_and(start < base + _TILE, end > base))
        def _apply(e=e, start=start, end=end):
            m = jnp.logical_and(rows >= start, rows < end)
            part = jnp.dot(xbf, w_ref[e], preferred_element_type=jnp.float32)
            out_ref[...] = jnp.where(m, part + b_ref[e], out_ref[...])


def _moe_call(offs, xs, w, b):
    grid_spec = pltpu.PrefetchScalarGridSpec(
        num_scalar_prefetch=1,
        grid=(_NT,),
        in_specs=[
            pl.BlockSpec((_TILE, _H), lambda t, offs: (t, 0)),
            pl.BlockSpec((_E, _H, _H), lambda t, offs: (0, 0, 0)),
            pl.BlockSpec((_E, _H), lambda t, offs: (0, 0)),
        ],
        out_specs=pl.BlockSpec((_TILE, _H), lambda t, offs: (t, 0)),
    )
    return pl.pallas_call(
        _moe_body,
        grid_spec=grid_spec,
        out_shape=jax.ShapeDtypeStruct((_N, _H), jnp.float32),
    )(offs, xs, w, b)


def _combine_body(y_hbm, pos_hbm, out_hbm, pos_v, rows_v, sem):
    wid = lax.axis_index("s") * _NC + lax.axis_index("c")
    base = wid * _TPW
    b, s0 = wid // _WPB, (wid % _WPB) * _TPW
    pltpu.sync_copy(pos_hbm.at[pl.ds(base, _TPW)], pos_v)
    pltpu.async_copy(y_hbm.at[pos_v], rows_v, sem).wait()
    pltpu.sync_copy(rows_v, out_hbm.at[b, pl.ds(s0, _TPW)])


@functools.cache
def _combine_call():
    return pl.kernel(
        _combine_body,
        out_type=jax.ShapeDtypeStruct((_B, _S, _H), jnp.float32),
        mesh=plsc.VectorSubcoreMesh(core_axis_name="c", subcore_axis_name="s"),
        scratch_types=[
            pltpu.VMEM((_TPW,), jnp.int32),
            pltpu.VMEM((_TPW, _H), jnp.float32),
            pltpu.SemaphoreType.DMA,
        ],
    )


def kernel(hidden_states, W1, b1, W2, b2, expert_W, expert_b):
    eid3, rank3, offs_f = _router_call(
        hidden_states, W1, b1.reshape(1, -1), W2, b2.reshape(1, -1))
    offs = offs_f.reshape(_E).astype(jnp.int32)
    offs_b = jnp.repeat(offs, _L)
    eid = eid3.reshape(_N)
    rank = rank3.reshape(_N)
    perm_x, pos = _dispatch_call()(hidden_states, eid, rank, offs_b)
    y = _moe_call(offs, perm_x, expert_W.astype(jnp.bfloat16), expert_b)
    return _combine_call()(y, pos)


# speedup vs baseline: 1.0393x; 1.0241x over previous
import functools

import jax
import jax.numpy as jnp
from jax import lax
from jax.experimental import pallas as pl
from jax.experimental.pallas import tpu as pltpu
from jax.experimental.pallas import tpu_sc as plsc

_B, _S, _H, _E = 2, 2048, 768, 8
_N = _B * _S
_TILE = 512
_NT = _N // _TILE
_T1 = 1024
_NT1 = _N // _T1
_NC, _NS, _L = 2, 16, 16
_NW = _NC * _NS
_TPW = _N // _NW
_WPB = _S // _TPW


def _router_body(x_ref, w1_ref, b1_ref, w2_ref, b2_ref,
                 eid_ref, rank_ref, offs_ref, run_ref, triu_ref):
    t = pl.program_id(0)

    @pl.when(t == 0)
    def _init():
        run_ref[...] = jnp.zeros_like(run_ref)
        ri = lax.broadcasted_iota(jnp.int32, (_T1, _T1), 0)
        ci = lax.broadcasted_iota(jnp.int32, (_T1, _T1), 1)
        triu_ref[...] = (ri < ci).astype(jnp.float32)

    x = x_ref[0]
    h = jnp.dot(x, w1_ref[...], preferred_element_type=jnp.float32)
    h = jnp.maximum(h + b1_ref[...], 0.0)
    s = jnp.dot(h, w2_ref[...], preferred_element_type=jnp.float32)
    s = s + b2_ref[...]

    st = s.T
    smax = jnp.max(st, axis=0, keepdims=True)
    rowi = lax.broadcasted_iota(jnp.int32, (_E, _T1), 0)
    eid = jnp.min(jnp.where(st == smax, rowi, _E), axis=0)
    eid_ref[0, 0, :] = eid

    onehot = (rowi == eid[None, :]).astype(jnp.float32)
    excl = jnp.dot(onehot, triu_ref[...], preferred_element_type=jnp.float32)
    run = run_ref[...]
    rank = jnp.sum(onehot * (excl + run), axis=0)
    rank_ref[0, 0, :] = rank.astype(jnp.int32)

    counts = jnp.dot(onehot, jnp.ones((_T1, 1), jnp.float32),
                     preferred_element_type=jnp.float32)
    new_run = run + counts
    run_ref[...] = new_run

    er = lax.broadcasted_iota(jnp.int32, (_E, _E), 0)
    ec = lax.broadcasted_iota(jnp.int32, (_E, _E), 1)
    ltm = (ec < er).astype(jnp.float32)
    offs = jnp.dot(ltm, new_run, preferred_element_type=jnp.float32,
                   precision=lax.Precision.HIGHEST)
    offs_ref[...] = offs


def _router_call(hs, w1, b1, w2, b2):
    return pl.pallas_call(
        _router_body,
        grid=(_NT1,),
        in_specs=[
            pl.BlockSpec((1, _T1, _H), lambda t: (t // (_S // _T1),
                                                  t % (_S // _T1), 0)),
            pl.BlockSpec((_H, _H // 2), lambda t: (0, 0)),
            pl.BlockSpec((1, _H // 2), lambda t: (0, 0)),
            pl.BlockSpec((_H // 2, _E), lambda t: (0, 0)),
            pl.BlockSpec((1, _E), lambda t: (0, 0)),
        ],
        out_specs=[
            pl.BlockSpec((1, 1, _T1), lambda t: (t, 0, 0)),
            pl.BlockSpec((1, 1, _T1), lambda t: (t, 0, 0)),
            pl.BlockSpec((_E, 1), lambda t: (0, 0)),
        ],
        out_shape=[
            jax.ShapeDtypeStruct((_NT1, 1, _T1), jnp.int32),
            jax.ShapeDtypeStruct((_NT1, 1, _T1), jnp.int32),
            jax.ShapeDtypeStruct((_E, 1), jnp.float32),
        ],
        scratch_shapes=[pltpu.VMEM((_E, 1), jnp.float32),
                        pltpu.VMEM((_T1, _T1), jnp.float32)],
    )(hs, w1, b1, w2, b2)


def _dispatch_body(x_hbm, eid_hbm, rank_hbm, offs_hbm,
                   perm_hbm, pos_hbm,
                   eid_v, rank_v, offs_v, pos_v, rows_v, sem_rows, sem_idx):
    wid = lax.axis_index("s") * _NC + lax.axis_index("c")
    base = wid * _TPW
    b, s0 = wid // _WPB, (wid % _WPB) * _TPW
    rows_cp = pltpu.async_copy(x_hbm.at[b, pl.ds(s0, _TPW)], rows_v, sem_rows)
    cp_e = pltpu.async_copy(eid_hbm.at[pl.ds(base, _TPW)], eid_v, sem_idx)
    cp_r = pltpu.async_copy(rank_hbm.at[pl.ds(base, _TPW)], rank_v, sem_idx)
    cp_o = pltpu.async_copy(offs_hbm, offs_v, sem_idx)
    cp_e.wait()
    cp_r.wait()
    cp_o.wait()
    offs_s = [offs_v[pl.ds(e * _L, _L)] for e in range(_E)]
    for i in range(_TPW // _L):
        ev = eid_v[pl.ds(i * _L, _L)]
        pos = rank_v[pl.ds(i * _L, _L)]
        for e in range(_E):
            pos = jnp.where(ev == e, pos + offs_s[e], pos)
        pos_v[pl.ds(i * _L, _L)] = pos
    pltpu.sync_copy(pos_v, pos_hbm.at[pl.ds(base, _TPW)])
    rows_cp.wait()
    pltpu.sync_copy(rows_v, perm_hbm.at[pos_v])


@functools.cache
def _dispatch_call():
    return pl.kernel(
        _dispatch_body,
        out_type=[
            jax.ShapeDtypeStruct((_N, _H), jnp.float32),
            jax.ShapeDtypeStruct((_N,), jnp.int32),
        ],
        mesh=plsc.VectorSubcoreMesh(core_axis_name="c", subcore_axis_name="s"),
        scratch_types=[
            pltpu.VMEM((_TPW,), jnp.int32),
            pltpu.VMEM((_TPW,), jnp.int32),
            pltpu.VMEM((_E * _L,), jnp.int32),
            pltpu.VMEM((_TPW,), jnp.int32),
            pltpu.VMEM((_TPW, _H), jnp.float32),
            pltpu.SemaphoreType.DMA,
            pltpu.SemaphoreType.DMA,
        ],
    )


def _moe_body(offs_ref, x_ref, w_ref, b_ref, out_ref):
    t = pl.program_id(0)
    base = t * _TILE
    rows = base + lax.broadcasted_iota(jnp.int32, (_TILE, 1), 0)
    xbf = x_ref[...].astype(jnp.bfloat16)
    for e in range(_E):
        start = offs_ref[e]
        end = offs_ref[e + 1] if e + 1 < _E else _N

        @pl.when(jnp.logical_and(start < base + _TILE, end > base))
        def _apply(e=e, start=start, end=end):
            m = jnp.logical_and(rows >= start, rows < end)
            part = jnp.dot(xbf, w_ref[e], preferred_element_type=jnp.float32)
            out_ref[...] = jnp.where(m, part + b_ref[e], out_ref[...])


def _moe_call(offs, xs, w, b):
    grid_spec = pltpu.PrefetchScalarGridSpec(
        num_scalar_prefetch=1,
        grid=(_NT,),
        in_specs=[
            pl.BlockSpec((_TILE, _H), lambda t, offs: (t, 0)),
            pl.BlockSpec((_E, _H, _H), lambda t, offs: (0, 0, 0)),
            pl.BlockSpec((_E, _H), lambda t, offs: (0, 0)),
        ],
        out_specs=pl.BlockSpec((_TILE, _H), lambda t, offs: (t, 0)),
    )
    return pl.pallas_call(
        _moe_body,
        grid_spec=grid_spec,
        out_shape=jax.ShapeDtypeStruct((_N, _H), jnp.float32),
    )(offs, xs, w, b)


def _combine_body(y_hbm, pos_hbm, out_hbm, pos_v, rows_v, sem):
    wid = lax.axis_index("s") * _NC + lax.axis_index("c")
    base = wid * _TPW
    b, s0 = wid // _WPB, (wid % _WPB) * _TPW
    pltpu.sync_copy(pos_hbm.at[pl.ds(base, _TPW)], pos_v)
    pltpu.async_copy(y_hbm.at[pos_v], rows_v, sem).wait()
    pltpu.sync_copy(rows_v, out_hbm.at[b, pl.ds(s0, _TPW)])


@functools.cache
def _combine_call():
    return pl.kernel(
        _combine_body,
        out_type=jax.ShapeDtypeStruct((_B, _S, _H), jnp.float32),
        mesh=plsc.VectorSubcoreMesh(core_axis_name="c", subcore_axis_name="s"),
        scratch_types=[
            pltpu.VMEM((_TPW,), jnp.int32),
            pltpu.VMEM((_TPW, _H), jnp.float32),
            pltpu.SemaphoreType.DMA,
        ],
    )


def kernel(hidden_states, W1, b1, W2, b2, expert_W, expert_b):
    eid3, rank3, offs_f = _router_call(
        hidden_states, W1, b1.reshape(1, -1), W2, b2.reshape(1, -1))
    offs = offs_f.reshape(_E).astype(jnp.int32)
    offs_b = jnp.repeat(offs, _L)
    eid = eid3.reshape(_N)
    rank = rank3.reshape(_N)
    perm_x, pos = _dispatch_call()(hidden_states, eid, rank, offs_b)
    y = _moe_call(offs, perm_x, expert_W.astype(jnp.bfloat16), expert_b)
    return _combine_call()(y, pos)
